# Initial kernel scaffold; baseline (speedup 1.0000x reference)
import jax
import jax.numpy as jnp
from jax.experimental import pallas as pl


def kernel(input, adj, nbr_idx, weight_0, weights_mask0, bias):
    raise NotImplementedError("write your pallas kernel here")


# speedup vs baseline: 1.7737x; 1.7737x over previous
import functools

import jax
import jax.numpy as jnp
from jax import lax
from jax.experimental import pallas as pl
from jax.experimental.pallas import tpu as pltpu
from jax.experimental.pallas import tpu_sc as plsc

N = 10000
D = 128
DEG = 32

NC = 2
NS = 16
NW = NC * NS
NPW = 320
NP = NW * NPW
C = 4
E = C * DEG
NCHUNK = NPW // C
NBUF = 2
NV = D // 16


def _sc_aggregate(inp_pad, nbr_flat, asc, bsc):
    mesh = plsc.VectorSubcoreMesh(
        core_axis_name="c", subcore_axis_name="s", num_cores=NC, num_subcores=NS)

    @functools.partial(
        pl.kernel,
        out_type=jax.ShapeDtypeStruct((NP, D), jnp.float32),
        mesh=mesh,
        compiler_params=pltpu.CompilerParams(needs_layout_passes=False),
        scratch_types=[
            pltpu.VMEM((NP,), jnp.float32),
            pltpu.VMEM((NPW + 16,), jnp.float32),
            pltpu.VMEM((NBUF, E), jnp.int32),
            pltpu.VMEM((NBUF, E, D), jnp.float32),
            pltpu.VMEM((NBUF, C, D), jnp.float32),
            pltpu.VMEM((E,), jnp.float32),
            pltpu.SemaphoreType.DMA((NBUF,)),
            pltpu.SemaphoreType.DMA((NBUF,)),
        ],
    )
    def k(inp_hbm, nbr_hbm, asc_hbm, bsc_hbm, out_hbm,
          bsc_v, asc_v, idx_v, rows_v, acc_v, mask_v, gsem, csem):
        wid = lax.axis_index("s") * NC + lax.axis_index("c")
        nbase = wid * NPW
        pltpu.sync_copy(bsc_hbm, bsc_v)
        pltpu.sync_copy(asc_hbm.at[pl.ds(nbase, NPW)], asc_v.at[pl.ds(0, NPW)])

        def start(gc, b):
            node0 = nbase + gc * C
            pltpu.sync_copy(nbr_hbm.at[pl.ds(node0 * DEG, E)], idx_v.at[b])
            pltpu.async_copy(inp_hbm.at[idx_v.at[b]], rows_v.at[b], gsem.at[b])
            pltpu.async_copy(inp_hbm.at[pl.ds(node0, C)], acc_v.at[b], csem.at[b])

        for b in range(NBUF):
            start(b, b)

        @pl.loop(0, NCHUNK, step=NBUF)
        def _outer(g0):
            for b in range(NBUF):
                gc = g0 + b
                node0 = nbase + gc * C
                pltpu.make_async_copy(
                    inp_hbm.at[idx_v.at[b]], rows_v.at[b], gsem.at[b]).wait()
                pltpu.make_async_copy(
                    inp_hbm.at[pl.ds(node0, C)], acc_v.at[b], csem.at[b]).wait()
                av = asc_v[pl.ds(gc * C, 16)]
                for v in range(E // 16):
                    idx16 = idx_v[b, pl.ds(v * 16, 16)]
                    bs = plsc.load_gather(bsc_v, [idx16])
                    x = bs + lax.broadcast(av[v // 2], (16,))
                    z = jnp.exp(-jnp.abs(x))
                    mask_v[pl.ds(v * 16, 16)] = jnp.where(
                        x >= 0, 1.0 / (1.0 + z), z / (1.0 + z))
                @pl.loop(0, C)
                def _node(n):
                    acc = [acc_v[b, n, pl.ds(v * 16, 16)] for v in range(NV)]
                    for h in range(DEG // 16):
                        mv = mask_v[pl.ds(n * DEG + h * 16, 16)]
                        for j in range(16):
                            e = n * DEG + h * 16 + j
                            m = lax.broadcast(mv[j], (16,))
                            for v in range(NV):
                                acc[v] = acc[v] + m * rows_v[b, e, pl.ds(v * 16, 16)]
                    for v in range(NV):
                        acc_v[b, n, pl.ds(v * 16, 16)] = acc[v]

                pltpu.sync_copy(acc_v.at[b], out_hbm.at[pl.ds(node0, C)])
                nxt = gc + NBUF

                @pl.when(nxt < NCHUNK)
                def _():
                    start(nxt, b)

    return k(inp_pad, nbr_flat, asc, bsc)


def _scores_matmul(inp_pad, wm_pad):
    blk = 1280

    def body(x_ref, w_ref, o_ref):
        o_ref[...] = jnp.dot(x_ref[...], w_ref[...],
                             preferred_element_type=jnp.float32)

    return pl.pallas_call(
        body,
        grid=(NP // blk,),
        in_specs=[
            pl.BlockSpec((blk, D), lambda i: (i, 0)),
            pl.BlockSpec((D, 128), lambda i: (0, 0)),
        ],
        out_specs=pl.BlockSpec((blk, 128), lambda i: (i, 0)),
        out_shape=jax.ShapeDtypeStruct((NP, 128), jnp.float32),
    )(inp_pad, wm_pad)


def _support_matmul(x, w):
    blk = 1000

    def body(x_ref, w_ref, o_ref):
        o_ref[...] = jnp.dot(x_ref[...], w_ref[...],
                             preferred_element_type=jnp.float32)

    return pl.pallas_call(
        body,
        grid=(N // blk,),
        in_specs=[
            pl.BlockSpec((blk, D), lambda i: (i, 0)),
            pl.BlockSpec((D, D), lambda i: (0, 0)),
        ],
        out_specs=pl.BlockSpec((blk, D), lambda i: (i, 0)),
        out_shape=jax.ShapeDtypeStruct((N, D), jnp.float32),
    )(x, w)


def _adj_matmul(adj, sup, bias_row):
    bm = 200

    def body(a_ref, s_ref, b_ref, o_ref):
        o_ref[...] = jnp.dot(a_ref[...], s_ref[...],
                             preferred_element_type=jnp.float32) + b_ref[...]

    return pl.pallas_call(
        body,
        grid=(N // bm,),
        in_specs=[
            pl.BlockSpec((bm, N), lambda i: (i, 0)),
            pl.BlockSpec((N, D), lambda i: (0, 0)),
            pl.BlockSpec((1, D), lambda i: (0, 0)),
        ],
        out_specs=pl.BlockSpec((bm, D), lambda i: (i, 0)),
        out_shape=jax.ShapeDtypeStruct((N, D), jnp.float32),
    )(adj, sup, bias_row)


def kernel(input, adj, nbr_idx, weight_0, weights_mask0, bias):
    inp = input.astype(jnp.float32)
    nbr = nbr_idx.astype(jnp.int32)

    inp_pad = jnp.zeros((NP, D), jnp.float32).at[:N].set(inp)
    nbr_flat = jnp.zeros((NP, DEG), jnp.int32).at[:N].set(nbr).reshape(NP * DEG)
    wm = weights_mask0.astype(jnp.float32).reshape(2 * D)
    wm_pad = jnp.zeros((D, 128), jnp.float32).at[:, 0].set(wm[:D]).at[:, 1].set(wm[D:])

    scores = _scores_matmul(inp_pad, wm_pad)
    asc = scores[:, 0]
    bsc = scores[:, 1]

    input_new = _sc_aggregate(inp_pad, nbr_flat, asc, bsc)
    sup = _support_matmul(input_new[:N], weight_0.astype(jnp.float32))
    return _adj_matmul(adj.astype(jnp.float32), sup,
                       bias.astype(jnp.float32).reshape(1, D))


# speedup vs baseline: 1.7810x; 1.0041x over previous
import functools

import jax
import jax.numpy as jnp
from jax import lax
from jax.experimental import pallas as pl
from jax.experimental.pallas import tpu as pltpu
from jax.experimental.pallas import tpu_sc as plsc

N = 10000
D = 128
DEG = 32

NC = 2
NS = 16
NW = NC * NS
NPW = 320
NP = NW * NPW
C = 4
E = C * DEG
NCHUNK = NPW // C
NBUF = 4
NV = D // 16


def _sc_aggregate(inp_pad, nbr_chunks, asc, bsc):
    mesh = plsc.VectorSubcoreMesh(
        core_axis_name="c", subcore_axis_name="s", num_cores=NC, num_subcores=NS)

    @functools.partial(
        pl.kernel,
        out_type=jax.ShapeDtypeStruct((NP, D), jnp.float32),
        mesh=mesh,
        compiler_params=pltpu.CompilerParams(needs_layout_passes=False),
        scratch_types=[
            pltpu.VMEM((NP,), jnp.float32),
            pltpu.VMEM((NPW + 16,), jnp.float32),
            pltpu.VMEM((NCHUNK, E), jnp.int32),
            pltpu.VMEM((NBUF, E, D), jnp.float32),
            pltpu.VMEM((NPW, D), jnp.float32),
            pltpu.VMEM((E,), jnp.float32),
            pltpu.SemaphoreType.DMA((NBUF,)),
            pltpu.SemaphoreType.DMA,
        ],
    )
    def k(inp_hbm, nbr_hbm, asc_hbm, bsc_hbm, out_hbm,
          bsc_v, asc_v, idx_v, rows_v, acc_v, mask_v, gsem, csem):
        wid = lax.axis_index("s") * NC + lax.axis_index("c")
        nbase = wid * NPW
        pltpu.sync_copy(nbr_hbm.at[wid], idx_v)
        pltpu.async_copy(inp_hbm.at[pl.ds(nbase, NPW)], acc_v, csem)
        pltpu.sync_copy(bsc_hbm, bsc_v)
        pltpu.sync_copy(asc_hbm.at[pl.ds(nbase, NPW)], asc_v.at[pl.ds(0, NPW)])

        def start(gc, b):
            pltpu.async_copy(inp_hbm.at[idx_v.at[gc]], rows_v.at[b], gsem.at[b])

        for b in range(NBUF):
            start(b, b)
        pltpu.make_async_copy(inp_hbm.at[pl.ds(nbase, NPW)], acc_v, csem).wait()

        @pl.loop(0, NCHUNK, step=NBUF)
        def _outer(g0):
            for b in range(NBUF):
                gc = g0 + b
                pltpu.make_async_copy(
                    inp_hbm.at[idx_v.at[gc]], rows_v.at[b], gsem.at[b]).wait()
                av = asc_v[pl.ds(gc * C, 16)]
                for v in range(E // 16):
                    idx16 = idx_v[gc, pl.ds(v * 16, 16)]
                    bs = plsc.load_gather(bsc_v, [idx16])
                    x = bs + lax.broadcast(av[v // 2], (16,))
                    z = jnp.exp(-jnp.abs(x))
                    mask_v[pl.ds(v * 16, 16)] = jnp.where(
                        x >= 0, 1.0 / (1.0 + z), z / (1.0 + z))
                @pl.loop(0, C)
                def _node(n):
                    node = gc * C + n
                    acc = [acc_v[node, pl.ds(v * 16, 16)] for v in range(NV)]
                    for h in range(DEG // 16):
                        mv = mask_v[pl.ds(n * DEG + h * 16, 16)]
                        for j in range(16):
                            e = n * DEG + h * 16 + j
                            m = lax.broadcast(mv[j], (16,))
                            for v in range(NV):
                                acc[v] = acc[v] + m * rows_v[b, e, pl.ds(v * 16, 16)]
                    for v in range(NV):
                        acc_v[node, pl.ds(v * 16, 16)] = acc[v]

                nxt = gc + NBUF

                @pl.when(nxt < NCHUNK)
                def _():
                    start(nxt, b)

        pltpu.sync_copy(acc_v, out_hbm.at[pl.ds(nbase, NPW)])

    return k(inp_pad, nbr_chunks, asc, bsc)


def _scores_matmul(inp_pad, wm_pad):
    blk = 1280

    def body(x_ref, w_ref, o_ref):
        o_ref[...] = jnp.dot(x_ref[...], w_ref[...],
                             preferred_element_type=jnp.float32)

    return pl.pallas_call(
        body,
        grid=(NP // blk,),
        in_specs=[
            pl.BlockSpec((blk, D), lambda i: (i, 0)),
            pl.BlockSpec((D, 128), lambda i: (0, 0)),
        ],
        out_specs=pl.BlockSpec((blk, 128), lambda i: (i, 0)),
        out_shape=jax.ShapeDtypeStruct((NP, 128), jnp.float32),
    )(inp_pad, wm_pad)


def _support_matmul(x, w):
    blk = 1000

    def body(x_ref, w_ref, o_ref):
        o_ref[...] = jnp.dot(x_ref[...], w_ref[...],
                             preferred_element_type=jnp.float32)

    return pl.pallas_call(
        body,
        grid=(N // blk,),
        in_specs=[
            pl.BlockSpec((blk, D), lambda i: (i, 0)),
            pl.BlockSpec((D, D), lambda i: (0, 0)),
        ],
        out_specs=pl.BlockSpec((blk, D), lambda i: (i, 0)),
        out_shape=jax.ShapeDtypeStruct((N, D), jnp.float32),
    )(x, w)


def _adj_matmul(adj, sup, bias_row):
    bm = 200

    def body(a_ref, s_ref, b_ref, o_ref):
        o_ref[...] = jnp.dot(a_ref[...], s_ref[...],
                             preferred_element_type=jnp.float32) + b_ref[...]

    return pl.pallas_call(
        body,
        grid=(N // bm,),
        in_specs=[
            pl.BlockSpec((bm, N), lambda i: (i, 0)),
            pl.BlockSpec((N, D), lambda i: (0, 0)),
            pl.BlockSpec((1, D), lambda i: (0, 0)),
        ],
        out_specs=pl.BlockSpec((bm, D), lambda i: (i, 0)),
        out_shape=jax.ShapeDtypeStruct((N, D), jnp.float32),
    )(adj, sup, bias_row)


def kernel(input, adj, nbr_idx, weight_0, weights_mask0, bias):
    inp = input.astype(jnp.float32)
    nbr = nbr_idx.astype(jnp.int32)

    inp_pad = jnp.zeros((NP, D), jnp.float32).at[:N].set(inp)
    nbr_chunks = (jnp.zeros((NP, DEG), jnp.int32).at[:N].set(nbr)
                  .reshape(NW, NCHUNK, E))
    wm = weights_mask0.astype(jnp.float32).reshape(2 * D)
    wm_pad = jnp.zeros((D, 128), jnp.float32).at[:, 0].set(wm[:D]).at[:, 1].set(wm[D:])

    scores = _scores_matmul(inp_pad, wm_pad)
    asc = scores[:, 0]
    bsc = scores[:, 1]

    input_new = _sc_aggregate(inp_pad, nbr_chunks, asc, bsc)
    sup = _support_matmul(input_new[:N], weight_0.astype(jnp.float32))
    return _adj_matmul(adj.astype(jnp.float32), sup,
                       bias.astype(jnp.float32).reshape(1, D))


# speedup vs baseline: 4.0898x; 2.2963x over previous
import functools

import jax
import jax.numpy as jnp
from jax import lax
from jax.experimental import pallas as pl
from jax.experimental.pallas import tpu as pltpu
from jax.experimental.pallas import tpu_sc as plsc

N = 10000
D = 128
DEG = 32

NC = 2
NS = 16
NW = NC * NS
NPW = 320
NP = NW * NPW
C = 4
E = C * DEG
NCHUNK = NPW // C
NBUF = 2
RING = 4
NV = D // 16
TROWS = NP // NS


def _sc_aggregate(inp_pad, nbr_chunks, asc, bsc):
    mesh = plsc.VectorSubcoreMesh(
        core_axis_name="c", subcore_axis_name="s", num_cores=NC, num_subcores=NS)

    @functools.partial(
        pl.kernel,
        out_type=jax.ShapeDtypeStruct((NP, D), jnp.float32),
        mesh=mesh,
        compiler_params=pltpu.CompilerParams(needs_layout_passes=False),
        scratch_types=[
            pltpu.VMEM((NP,), jnp.float32),
            pltpu.VMEM((NPW + 16,), jnp.float32),
            pltpu.VMEM((RING, E), jnp.int32),
            pltpu.VMEM((NBUF, E, D), jnp.float32),
            pltpu.VMEM((RING, C, D), jnp.float32),
            pltpu.VMEM((E,), jnp.float32),
            pltpu.VMEM_SHARED((NP, D), jnp.float32),
            pltpu.SemaphoreType.DMA((RING,)),
            pltpu.SemaphoreType.DMA((NBUF,)),
            pltpu.SemaphoreType.DMA((RING,)),
            pltpu.SemaphoreType.DMA,
        ],
    )
    def k(inp_hbm, nbr_hbm, asc_hbm, bsc_hbm, out_hbm,
          bsc_v, asc_v, idx_v, rows_v, acc_v, mask_v, tab_s,
          isem, gsem, osem, tsem):
        sid = lax.axis_index("s")
        wid = sid * NC + lax.axis_index("c")
        nbase = wid * NPW
        tb = sid * TROWS
        pltpu.async_copy(inp_hbm.at[pl.ds(tb, TROWS)],
                         tab_s.at[pl.ds(tb, TROWS)], tsem)
        for g in range(RING):
            pltpu.async_copy(nbr_hbm.at[wid, g], idx_v.at[g], isem.at[g])
        pltpu.sync_copy(bsc_hbm, bsc_v)
        pltpu.sync_copy(asc_hbm.at[pl.ds(nbase, NPW)], asc_v.at[pl.ds(0, NPW)])
        pltpu.make_async_copy(inp_hbm.at[pl.ds(tb, TROWS)],
                              tab_s.at[pl.ds(tb, TROWS)], tsem).wait()
        plsc.subcore_barrier()

        for g in range(NBUF):
            pltpu.make_async_copy(nbr_hbm.at[wid, g], idx_v.at[g],
                                  isem.at[g]).wait()
            pltpu.async_copy(tab_s.at[idx_v.at[g]], rows_v.at[g], gsem.at[g])

        @pl.loop(0, NCHUNK, step=RING)
        def _outer(g0):
            for q in range(RING):
                gc = g0 + q
                b = q % NBUF
                pltpu.make_async_copy(
                    tab_s.at[idx_v.at[q]], rows_v.at[b], gsem.at[b]).wait()
                av = asc_v[pl.ds(gc * C, 16)]
                for v in range(E // 16):
                    idx16 = idx_v[q, pl.ds(v * 16, 16)]
                    bs = plsc.load_gather(bsc_v, [idx16])
                    x = bs + lax.broadcast(av[v // 2], (16,))
                    z = jnp.exp(-jnp.abs(x))
                    mask_v[pl.ds(v * 16, 16)] = jnp.where(
                        x >= 0, 1.0 / (1.0 + z), z / (1.0 + z))
                @pl.when(gc + RING < NCHUNK)
                def _():
                    pltpu.async_copy(nbr_hbm.at[wid, gc + RING], idx_v.at[q],
                                     isem.at[q])
                @pl.when(gc >= RING)
                def _():
                    pltpu.make_async_copy(
                        acc_v.at[q],
                        out_hbm.at[pl.ds(nbase + (gc - RING) * C, C)],
                        osem.at[q]).wait()

                @pl.loop(0, C)
                def _node(n):
                    acc = [jnp.zeros((16,), jnp.float32) for _ in range(NV)]
                    for h in range(DEG // 16):
                        mv = mask_v[pl.ds(n * DEG + h * 16, 16)]
                        for j in range(16):
                            e = n * DEG + h * 16 + j
                            m = lax.broadcast(mv[j], (16,))
                            for v in range(NV):
                                acc[v] = acc[v] + m * rows_v[b, e, pl.ds(v * 16, 16)]
                    for v in range(NV):
                        acc_v[q, n, pl.ds(v * 16, 16)] = acc[v]

                pltpu.async_copy(acc_v.at[q],
                                 out_hbm.at[pl.ds(nbase + gc * C, C)], osem.at[q])
                @pl.when(gc + NBUF < NCHUNK)
                def _():
                    q2 = (q + NBUF) % RING
                    pltpu.make_async_copy(nbr_hbm.at[wid, gc + NBUF],
                                          idx_v.at[q2], isem.at[q2]).wait()
                    pltpu.async_copy(tab_s.at[idx_v.at[q2]], rows_v.at[b],
                                     gsem.at[b])

        for q in range(RING):
            pltpu.make_async_copy(
                acc_v.at[q],
                out_hbm.at[pl.ds(nbase + (NCHUNK - RING + q) * C, C)],
                osem.at[q]).wait()

    return k(inp_pad, nbr_chunks, asc, bsc)


def _scores_matmul(inp_pad, wm_pad):
    blk = 1280

    def body(x_ref, w_ref, o_ref):
        o_ref[...] = jnp.dot(x_ref[...], w_ref[...],
                             preferred_element_type=jnp.float32)

    return pl.pallas_call(
        body,
        grid=(NP // blk,),
        in_specs=[
            pl.BlockSpec((blk, D), lambda i: (i, 0)),
            pl.BlockSpec((D, 128), lambda i: (0, 0)),
        ],
        out_specs=pl.BlockSpec((blk, 128), lambda i: (i, 0)),
        out_shape=jax.ShapeDtypeStruct((NP, 128), jnp.float32),
    )(inp_pad, wm_pad)


def _support_matmul(x, agg, w):
    blk = 1000

    def body(x_ref, g_ref, w_ref, o_ref):
        o_ref[...] = jnp.dot(x_ref[...] + g_ref[...], w_ref[...],
                             preferred_element_type=jnp.float32)

    return pl.pallas_call(
        body,
        grid=(N // blk,),
        in_specs=[
            pl.BlockSpec((blk, D), lambda i: (i, 0)),
            pl.BlockSpec((blk, D), lambda i: (i, 0)),
            pl.BlockSpec((D, D), lambda i: (0, 0)),
        ],
        out_specs=pl.BlockSpec((blk, D), lambda i: (i, 0)),
        out_shape=jax.ShapeDtypeStruct((N, D), jnp.float32),
    )(x, agg, w)


def _adj_matmul(adj, sup, bias_row):
    bm = 200

    def body(a_ref, s_ref, b_ref, o_ref):
        o_ref[...] = jnp.dot(a_ref[...], s_ref[...],
                             preferred_element_type=jnp.float32) + b_ref[...]

    return pl.pallas_call(
        body,
        grid=(N // bm,),
        in_specs=[
            pl.BlockSpec((bm, N), lambda i: (i, 0)),
            pl.BlockSpec((N, D), lambda i: (0, 0)),
            pl.BlockSpec((1, D), lambda i: (0, 0)),
        ],
        out_specs=pl.BlockSpec((bm, D), lambda i: (i, 0)),
        out_shape=jax.ShapeDtypeStruct((N, D), jnp.float32),
    )(adj, sup, bias_row)


def kernel(input, adj, nbr_idx, weight_0, weights_mask0, bias):
    inp = input.astype(jnp.float32)
    nbr = nbr_idx.astype(jnp.int32)

    inp_pad = jnp.zeros((NP, D), jnp.float32).at[:N].set(inp)
    nbr_chunks = (jnp.zeros((NP, DEG), jnp.int32).at[:N].set(nbr)
                  .reshape(NW, NCHUNK, E))
    wm = weights_mask0.astype(jnp.float32).reshape(2 * D)
    wm_pad = jnp.zeros((D, 128), jnp.float32).at[:, 0].set(wm[:D]).at[:, 1].set(wm[D:])

    scores = _scores_matmul(inp_pad, wm_pad)
    asc = scores[:, 0]
    bsc = scores[:, 1]

    agg = _sc_aggregate(inp_pad, nbr_chunks, asc, bsc)
    sup = _support_matmul(inp, agg[:N], weight_0.astype(jnp.float32))
    return _adj_matmul(adj.astype(jnp.float32), sup,
                       bias.astype(jnp.float32).reshape(1, D))


# speedup vs baseline: 4.1385x; 1.0119x over previous
import functools

import jax
import jax.numpy as jnp
from jax import lax
from jax.experimental import pallas as pl
from jax.experimental.pallas import tpu as pltpu
from jax.experimental.pallas import tpu_sc as plsc

N = 10000
D = 128
DEG = 32

NC = 2
NS = 16
NW = NC * NS
NPW = 320
NP = NW * NPW
C = 4
E = C * DEG
NCHUNK = NPW // C
NBUF = 2
RING = 4
NV = D // 16
TROWS = 624


def _sc_aggregate(inp, nbr_chunks, asc, bsc):
    mesh = plsc.VectorSubcoreMesh(
        core_axis_name="c", subcore_axis_name="s", num_cores=NC, num_subcores=NS)

    @functools.partial(
        pl.kernel,
        out_type=jax.ShapeDtypeStruct((NP, D), jnp.float32),
        mesh=mesh,
        compiler_params=pltpu.CompilerParams(needs_layout_passes=False),
        scratch_types=[
            pltpu.VMEM((NP,), jnp.float32),
            pltpu.VMEM((NPW + 16,), jnp.float32),
            pltpu.VMEM((RING, E), jnp.int32),
            pltpu.VMEM((NBUF, E, D), jnp.float32),
            pltpu.VMEM((RING, C, D), jnp.float32),
            pltpu.VMEM((E,), jnp.float32),
            pltpu.VMEM_SHARED((N, D), jnp.float32),
            pltpu.SemaphoreType.DMA((RING,)),
            pltpu.SemaphoreType.DMA((NBUF,)),
            pltpu.SemaphoreType.DMA((RING,)),
            pltpu.SemaphoreType.DMA,
        ],
    )
    def k(inp_hbm, nbr_hbm, asc_hbm, bsc_hbm, out_hbm,
          bsc_v, asc_v, idx_v, rows_v, acc_v, mask_v, tab_s,
          isem, gsem, osem, tsem):
        sid = lax.axis_index("s")
        wid = sid * NC + lax.axis_index("c")
        nbase = wid * NPW
        tb = sid * TROWS
        pltpu.async_copy(inp_hbm.at[pl.ds(tb, TROWS)],
                         tab_s.at[pl.ds(tb, TROWS)], tsem)

        @pl.when(sid == 0)
        def _():
            pltpu.sync_copy(inp_hbm.at[pl.ds(NS * TROWS, N - NS * TROWS)],
                            tab_s.at[pl.ds(NS * TROWS, N - NS * TROWS)])

        for g in range(RING):
            pltpu.async_copy(nbr_hbm.at[wid, g], idx_v.at[g], isem.at[g])
        pltpu.sync_copy(bsc_hbm, bsc_v)
        pltpu.sync_copy(asc_hbm.at[pl.ds(nbase, NPW)], asc_v.at[pl.ds(0, NPW)])
        pltpu.make_async_copy(inp_hbm.at[pl.ds(tb, TROWS)],
                              tab_s.at[pl.ds(tb, TROWS)], tsem).wait()
        plsc.subcore_barrier()

        for g in range(NBUF):
            pltpu.make_async_copy(nbr_hbm.at[wid, g], idx_v.at[g],
                                  isem.at[g]).wait()
            pltpu.async_copy(tab_s.at[idx_v.at[g]], rows_v.at[g], gsem.at[g])

        @pl.loop(0, NCHUNK, step=RING)
        def _outer(g0):
            for q in range(RING):
                gc = g0 + q
                b = q % NBUF
                pltpu.make_async_copy(
                    tab_s.at[idx_v.at[q]], rows_v.at[b], gsem.at[b]).wait()
                av = asc_v[pl.ds(gc * C, 16)]
                for v in range(E // 16):
                    idx16 = idx_v[q, pl.ds(v * 16, 16)]
                    bs = plsc.load_gather(bsc_v, [idx16])
                    x = bs + lax.broadcast(av[v // 2], (16,))
                    z = jnp.exp(-jnp.abs(x))
                    mask_v[pl.ds(v * 16, 16)] = jnp.where(
                        x >= 0, 1.0 / (1.0 + z), z / (1.0 + z))
                @pl.when(gc + RING < NCHUNK)
                def _():
                    pltpu.async_copy(nbr_hbm.at[wid, gc + RING], idx_v.at[q],
                                     isem.at[q])
                @pl.when(gc >= RING)
                def _():
                    pltpu.make_async_copy(
                        acc_v.at[q],
                        out_hbm.at[pl.ds(nbase + (gc - RING) * C, C)],
                        osem.at[q]).wait()

                @pl.loop(0, C)
                def _node(n):
                    acc = [jnp.zeros((16,), jnp.float32) for _ in range(NV)]
                    for h in range(DEG // 16):
                        mv = mask_v[pl.ds(n * DEG + h * 16, 16)]
                        for j in range(16):
                            e = n * DEG + h * 16 + j
                            m = lax.broadcast(mv[j], (16,))
                            for v in range(NV):
                                acc[v] = acc[v] + m * rows_v[b, e, pl.ds(v * 16, 16)]
                    for v in range(NV):
                        acc_v[q, n, pl.ds(v * 16, 16)] = acc[v]

                pltpu.async_copy(acc_v.at[q],
                                 out_hbm.at[pl.ds(nbase + gc * C, C)], osem.at[q])
                @pl.when(gc + NBUF < NCHUNK)
                def _():
                    q2 = (q + NBUF) % RING
                    pltpu.make_async_copy(nbr_hbm.at[wid, gc + NBUF],
                                          idx_v.at[q2], isem.at[q2]).wait()
                    pltpu.async_copy(tab_s.at[idx_v.at[q2]], rows_v.at[b],
                                     gsem.at[b])

        for q in range(RING):
            pltpu.make_async_copy(
                acc_v.at[q],
                out_hbm.at[pl.ds(nbase + (NCHUNK - RING + q) * C, C)],
                osem.at[q]).wait()

    return k(inp, nbr_chunks, asc, bsc)


def _scores_matmul(inp, wm_pad):
    blk = 1000

    def body(x_ref, w_ref, o_ref):
        o_ref[...] = jnp.dot(x_ref[...], w_ref[...],
                             preferred_element_type=jnp.float32)

    return pl.pallas_call(
        body,
        grid=(N // blk,),
        in_specs=[
            pl.BlockSpec((blk, D), lambda i: (i, 0)),
            pl.BlockSpec((D, 8), lambda i: (0, 0)),
        ],
        out_specs=pl.BlockSpec((blk, 8), lambda i: (i, 0)),
        out_shape=jax.ShapeDtypeStruct((N, 8), jnp.float32),
    )(inp, wm_pad)


def _support_matmul(x, agg, w):
    blk = 1000

    def body(x_ref, g_ref, w_ref, o_ref):
        o_ref[...] = jnp.dot(x_ref[...] + g_ref[...], w_ref[...],
                             preferred_element_type=jnp.float32)

    return pl.pallas_call(
        body,
        grid=(N // blk,),
        in_specs=[
            pl.BlockSpec((blk, D), lambda i: (i, 0)),
            pl.BlockSpec((blk, D), lambda i: (i, 0)),
            pl.BlockSpec((D, D), lambda i: (0, 0)),
        ],
        out_specs=pl.BlockSpec((blk, D), lambda i: (i, 0)),
        out_shape=jax.ShapeDtypeStruct((N, D), jnp.float32),
    )(x, agg, w)


def _adj_matmul(adj, sup, bias_row):
    bm = 400

    def body(a_ref, s_ref, b_ref, o_ref):
        o_ref[...] = jnp.dot(a_ref[...], s_ref[...],
                             preferred_element_type=jnp.float32) + b_ref[...]

    return pl.pallas_call(
        body,
        grid=(N // bm,),
        in_specs=[
            pl.BlockSpec((bm, N), lambda i: (i, 0)),
            pl.BlockSpec((N, D), lambda i: (0, 0)),
            pl.BlockSpec((1, D), lambda i: (0, 0)),
        ],
        out_specs=pl.BlockSpec((bm, D), lambda i: (i, 0)),
        out_shape=jax.ShapeDtypeStruct((N, D), jnp.float32),
    )(adj, sup, bias_row)


def kernel(input, adj, nbr_idx, weight_0, weights_mask0, bias):
    inp = input.astype(jnp.float32)
    nbr = nbr_idx.astype(jnp.int32)

    nbr_chunks = (jnp.zeros((NP, DEG), jnp.int32).at[:N].set(nbr)
                  .reshape(NW, NCHUNK, E))
    wm = weights_mask0.astype(jnp.float32).reshape(2 * D)
    wm_pad = jnp.zeros((D, 8), jnp.float32).at[:, 0].set(wm[:D]).at[:, 1].set(wm[D:])

    scores = _scores_matmul(inp, wm_pad)
    pad8 = jnp.zeros((NP - N, 8), jnp.float32)
    scores_pad = jnp.concatenate([scores, pad8], axis=0)
    asc = scores_pad[:, 0]
    bsc = scores_pad[:, 1]

    agg = _sc_aggregate(inp, nbr_chunks, asc, bsc)
    sup = _support_matmul(inp, agg[:N], weight_0.astype(jnp.float32))
    return _adj_matmul(adj.astype(jnp.float32), sup,
                       bias.astype(jnp.float32).reshape(1, D))
